# baseline (speedup 1.0000x reference)
import jax
import jax.numpy as jnp
from jax.experimental import pallas as pl

_HID = 128
_OUT = 32
_NG = 16
_NSTEPS = 100
_R = 1024


def _stage_a(agg0_ref, x0_ref, batch_ref, temb_ref, w1_ref, b1_ref, p_ref,
             h1_ref, score_ref):
    s = agg0_ref[...] + x0_ref[...]
    h = jnp.maximum(s * w1_ref[...] + b1_ref[...], 0.0)
    iota = jax.lax.broadcasted_iota(jnp.int32, (_R, _NG), 1)
    oh = (batch_ref[...] == iota).astype(jnp.float32)
    h1 = h + jnp.dot(oh, temb_ref[...], preferred_element_type=jnp.float32)
    h1_ref[...] = h1
    score_ref[...] = jnp.dot(h1, p_ref[...], preferred_element_type=jnp.float32)


def _stage_b(h1_ref, gate_ref, w2_ref, out_ref):
    hp = h1_ref[...] * gate_ref[...]
    out_ref[...] = jnp.dot(hp, w2_ref[...], preferred_element_type=jnp.float32)


def _stage_c(aggb_ref, hpw2_ref, h1_ref, b2_ref, w3_ref, out_ref):
    h2 = jnp.maximum(aggb_ref[...] + hpw2_ref[...] + b2_ref[...], 0.0)
    h2 = h2 + h1_ref[...]
    out_ref[...] = jnp.dot(h2, w3_ref[...], preferred_element_type=jnp.float32)


def _sin_emb(t, dim):
    half = dim // 2
    freqs = jnp.exp(-jnp.log(10000.0) * jnp.arange(half, dtype=jnp.float32) / half)
    args = t.astype(jnp.float32)[:, None] * freqs[None, :]
    return jnp.concatenate([jnp.sin(args), jnp.cos(args)], axis=-1)


def _row_spec(cols):
    return pl.BlockSpec((_R, cols), lambda i: (i, 0))


def _full_spec(rows, cols):
    return pl.BlockSpec((rows, cols), lambda i: (0, 0))


@jax.jit
def kernel(x, edge_weight, W1, b1, Wt, bt, p, W2, b2, W3, b3,
           t, transmitters_index, edge_index, batch):
    nx = x.shape[0]
    nn = transmitters_index.shape[0]
    npad = ((nn + _R - 1) // _R) * _R
    grid = (npad // _R,)
    src, dst = edge_index[0], edge_index[1]

    x0 = jnp.take(x, transmitters_index, axis=0)
    x0s = x0[:, 0]

    agg0 = jax.ops.segment_sum(edge_weight * jnp.take(x0s, src), dst,
                               num_segments=nn)

    temb = _sin_emb(t, _HID) @ Wt + bt
    pnorm = p / (jnp.linalg.norm(p) + 1e-8)

    def padr(a):
        return jnp.pad(a, ((0, npad - nn), (0, 0)))

    h1p, scorep = pl.pallas_call(
        _stage_a,
        grid=grid,
        in_specs=[_row_spec(1), _row_spec(1), _row_spec(1),
                  _full_spec(_NG, _HID), _full_spec(1, _HID),
                  _full_spec(1, _HID), _full_spec(_HID, 1)],
        out_specs=[_row_spec(_HID), _row_spec(1)],
        out_shape=[jax.ShapeDtypeStruct((npad, _HID), jnp.float32),
                   jax.ShapeDtypeStruct((npad, 1), jnp.float32)],
    )(padr(agg0[:, None]), padr(x0), padr(batch[:, None].astype(jnp.int32)),
      temb, W1, b1[None, :], pnorm[:, None])

    score = scorep[:nn, 0]

    k = nn // 2
    _, idx = jax.lax.top_k(score, k)
    mask = jnp.zeros((nn,), dtype=jnp.float32).at[idx].set(1.0)
    gate = jax.nn.sigmoid(score) * mask

    hpw2p = pl.pallas_call(
        _stage_b,
        grid=grid,
        in_specs=[_row_spec(_HID), _row_spec(1), _full_spec(_HID, _HID)],
        out_specs=_row_spec(_HID),
        out_shape=jax.ShapeDtypeStruct((npad, _HID), jnp.float32),
    )(h1p, padr(gate[:, None]), W2)
    hpw2 = hpw2p[:nn]

    ewp = edge_weight * jnp.take(mask, src) * jnp.take(mask, dst)
    aggb = jax.ops.segment_sum(ewp[:, None] * jnp.take(hpw2, src, axis=0),
                               dst, num_segments=nn)

    hw3p = pl.pallas_call(
        _stage_c,
        grid=grid,
        in_specs=[_row_spec(_HID), _row_spec(_HID), _row_spec(_HID),
                  _full_spec(1, _HID), _full_spec(_HID, _OUT)],
        out_specs=_row_spec(_OUT),
        out_shape=jax.ShapeDtypeStruct((npad, _OUT), jnp.float32),
    )(padr(aggb), hpw2p, h1p, b2[None, :], W3)
    hw3 = hw3p[:nn]

    aggc = jax.ops.segment_sum(edge_weight[:, None] * jnp.take(hw3, src, axis=0),
                               dst, num_segments=nn)
    h3 = aggc + hw3 + b3

    sums = jax.ops.segment_sum(h3, transmitters_index, num_segments=nx)
    cnt = jax.ops.segment_sum(jnp.ones((nn,), dtype=jnp.float32),
                              transmitters_index, num_segments=nx)
    return sums / jnp.maximum(cnt, 1.0)[:, None]


# speedup vs baseline: 1.0039x; 1.0039x over previous
import jax
import jax.numpy as jnp
from jax.experimental import pallas as pl

_HID = 128
_OUT = 32
_NG = 16
_NSTEPS = 100
_R = 1000


def _stage_a(agg0_ref, x0_ref, batch_ref, temb_ref, w1_ref, b1_ref, p_ref,
             h1_ref, score_ref):
    s = agg0_ref[...] + x0_ref[...]
    h = jnp.maximum(s * w1_ref[...] + b1_ref[...], 0.0)
    iota = jax.lax.broadcasted_iota(jnp.int32, (_R, _NG), 1)
    oh = (batch_ref[...] == iota).astype(jnp.float32)
    h1 = h + jnp.dot(oh, temb_ref[...], preferred_element_type=jnp.float32)
    h1_ref[...] = h1
    score_ref[...] = jnp.dot(h1, p_ref[...], preferred_element_type=jnp.float32)


def _stage_b(h1_ref, gate_ref, w2_ref, out_ref):
    hp = h1_ref[...] * gate_ref[...]
    out_ref[...] = jnp.dot(hp, w2_ref[...], preferred_element_type=jnp.float32)


def _stage_c(aggb_ref, hpw2_ref, h1_ref, b2_ref, w3_ref, out_ref):
    h2 = jnp.maximum(aggb_ref[...] + hpw2_ref[...] + b2_ref[...], 0.0)
    h2 = h2 + h1_ref[...]
    out_ref[...] = jnp.dot(h2, w3_ref[...], preferred_element_type=jnp.float32)


def _sin_emb(t, dim):
    half = dim // 2
    freqs = jnp.exp(-jnp.log(10000.0) * jnp.arange(half, dtype=jnp.float32) / half)
    args = t.astype(jnp.float32)[:, None] * freqs[None, :]
    return jnp.concatenate([jnp.sin(args), jnp.cos(args)], axis=-1)


def _row_spec(cols):
    return pl.BlockSpec((_R, cols), lambda i: (i, 0))


def _full_spec(rows, cols):
    return pl.BlockSpec((rows, cols), lambda i: (0, 0))


@jax.jit
def kernel(x, edge_weight, W1, b1, Wt, bt, p, W2, b2, W3, b3,
           t, transmitters_index, edge_index, batch):
    nx = x.shape[0]
    nn = transmitters_index.shape[0]
    grid = (nn // _R,)
    src, dst = edge_index[0], edge_index[1]

    x0 = jnp.take(x, transmitters_index, axis=0)
    x0s = x0[:, 0]

    agg0 = jax.ops.segment_sum(edge_weight * jnp.take(x0s, src), dst,
                               num_segments=nn)

    temb = _sin_emb(t, _HID) @ Wt + bt
    pnorm = p / (jnp.linalg.norm(p) + 1e-8)

    h1p, scorep = pl.pallas_call(
        _stage_a,
        grid=grid,
        in_specs=[_row_spec(1), _row_spec(1), _row_spec(1),
                  _full_spec(_NG, _HID), _full_spec(1, _HID),
                  _full_spec(1, _HID), _full_spec(_HID, 1)],
        out_specs=[_row_spec(_HID), _row_spec(1)],
        out_shape=[jax.ShapeDtypeStruct((nn, _HID), jnp.float32),
                   jax.ShapeDtypeStruct((nn, 1), jnp.float32)],
    )(agg0[:, None], x0, batch[:, None].astype(jnp.int32),
      temb, W1, b1[None, :], pnorm[:, None])

    score = scorep[:, 0]

    k = nn // 2
    _, idx = jax.lax.top_k(score, k)
    mask = jnp.zeros((nn,), dtype=jnp.float32).at[idx].set(1.0)
    gate = jax.nn.sigmoid(score) * mask

    hpw2 = pl.pallas_call(
        _stage_b,
        grid=grid,
        in_specs=[_row_spec(_HID), _row_spec(1), _full_spec(_HID, _HID)],
        out_specs=_row_spec(_HID),
        out_shape=jax.ShapeDtypeStruct((nn, _HID), jnp.float32),
    )(h1p, gate[:, None], W2)

    ewp = edge_weight * jnp.take(mask, src) * jnp.take(mask, dst)
    aggb = jax.ops.segment_sum(ewp[:, None] * jnp.take(hpw2, src, axis=0),
                               dst, num_segments=nn)

    hw3 = pl.pallas_call(
        _stage_c,
        grid=grid,
        in_specs=[_row_spec(_HID), _row_spec(_HID), _row_spec(_HID),
                  _full_spec(1, _HID), _full_spec(_HID, _OUT)],
        out_specs=_row_spec(_OUT),
        out_shape=jax.ShapeDtypeStruct((nn, _OUT), jnp.float32),
    )(aggb, hpw2, h1p, b2[None, :], W3)

    aggc = jax.ops.segment_sum(edge_weight[:, None] * jnp.take(hw3, src, axis=0),
                               dst, num_segments=nn)
    h3 = aggc + hw3 + b3

    sums = jax.ops.segment_sum(h3, transmitters_index, num_segments=nx)
    cnt = jax.ops.segment_sum(jnp.ones((nn,), dtype=jnp.float32),
                              transmitters_index, num_segments=nx)
    return sums / jnp.maximum(cnt, 1.0)[:, None]


# speedup vs baseline: 1.1704x; 1.1659x over previous
import jax
import jax.numpy as jnp
from jax.experimental import pallas as pl

_HID = 128
_OUT = 32
_NG = 16
_NSTEPS = 100
_R = 1000


def _stage_a(agg0_ref, x0_ref, batch_ref, temb_ref, w1_ref, b1_ref, p_ref,
             h1_ref, score_ref):
    s = agg0_ref[...] + x0_ref[...]
    h = jnp.maximum(s * w1_ref[...] + b1_ref[...], 0.0)
    iota = jax.lax.broadcasted_iota(jnp.int32, (_R, _NG), 1)
    oh = (batch_ref[...] == iota).astype(jnp.float32)
    h1 = h + jnp.dot(oh, temb_ref[...], preferred_element_type=jnp.float32)
    h1_ref[...] = h1
    score_ref[...] = jnp.dot(h1, p_ref[...], preferred_element_type=jnp.float32)


def _stage_b(h1_ref, gate_ref, w2_ref, out_ref):
    hp = h1_ref[...] * gate_ref[...]
    out_ref[...] = jnp.dot(hp, w2_ref[...], preferred_element_type=jnp.float32)


def _stage_c(aggb_ref, hpw2_ref, h1_ref, b2_ref, w3_ref, out_ref):
    h2 = jnp.maximum(aggb_ref[...] + hpw2_ref[...] + b2_ref[...], 0.0)
    h2 = h2 + h1_ref[...]
    out_ref[...] = jnp.dot(h2, w3_ref[...], preferred_element_type=jnp.float32)


def _sin_emb(t, dim):
    half = dim // 2
    freqs = jnp.exp(-jnp.log(10000.0) * jnp.arange(half, dtype=jnp.float32) / half)
    args = t.astype(jnp.float32)[:, None] * freqs[None, :]
    return jnp.concatenate([jnp.sin(args), jnp.cos(args)], axis=-1)


def _row_spec(cols):
    return pl.BlockSpec((_R, cols), lambda i: (i, 0))


def _full_spec(rows, cols):
    return pl.BlockSpec((rows, cols), lambda i: (0, 0))


@jax.jit
def kernel(x, edge_weight, W1, b1, Wt, bt, p, W2, b2, W3, b3,
           t, transmitters_index, edge_index, batch):
    nx = x.shape[0]
    nn = transmitters_index.shape[0]
    grid = (nn // _R,)
    src, dst = edge_index[0], edge_index[1]

    x0 = jnp.take(x, transmitters_index, axis=0)

    x08 = jnp.broadcast_to(x0, (nn, 8))
    g8 = edge_weight[:, None] * jnp.take(x08, src, axis=0)
    agg0 = jax.ops.segment_sum(g8, dst, num_segments=nn)[:, :1]

    temb = _sin_emb(t, _HID) @ Wt + bt
    pnorm = p / (jnp.linalg.norm(p) + 1e-8)

    h1p, scorep = pl.pallas_call(
        _stage_a,
        grid=grid,
        in_specs=[_row_spec(1), _row_spec(1), _row_spec(1),
                  _full_spec(_NG, _HID), _full_spec(1, _HID),
                  _full_spec(1, _HID), _full_spec(_HID, 1)],
        out_specs=[_row_spec(_HID), _row_spec(1)],
        out_shape=[jax.ShapeDtypeStruct((nn, _HID), jnp.float32),
                   jax.ShapeDtypeStruct((nn, 1), jnp.float32)],
    )(agg0, x0, batch[:, None].astype(jnp.int32),
      temb, W1, b1[None, :], pnorm[:, None])

    score = scorep[:, 0]

    k = nn // 2
    _, idx = jax.lax.top_k(score, k)
    mask = jnp.zeros((nn,), dtype=jnp.float32).at[idx].set(1.0)
    gate = jax.nn.sigmoid(score) * mask

    hpw2 = pl.pallas_call(
        _stage_b,
        grid=grid,
        in_specs=[_row_spec(_HID), _row_spec(1), _full_spec(_HID, _HID)],
        out_specs=_row_spec(_HID),
        out_shape=jax.ShapeDtypeStruct((nn, _HID), jnp.float32),
    )(h1p, gate[:, None], W2)

    ewp = edge_weight * jnp.take(mask, src) * jnp.take(mask, dst)
    aggb = jax.ops.segment_sum(ewp[:, None] * jnp.take(hpw2, src, axis=0),
                               dst, num_segments=nn)

    hw3 = pl.pallas_call(
        _stage_c,
        grid=grid,
        in_specs=[_row_spec(_HID), _row_spec(_HID), _row_spec(_HID),
                  _full_spec(1, _HID), _full_spec(_HID, _OUT)],
        out_specs=_row_spec(_OUT),
        out_shape=jax.ShapeDtypeStruct((nn, _OUT), jnp.float32),
    )(aggb, hpw2, h1p, b2[None, :], W3)

    aggc = jax.ops.segment_sum(edge_weight[:, None] * jnp.take(hw3, src, axis=0),
                               dst, num_segments=nn)
    h3 = aggc + hw3 + b3

    sums = jax.ops.segment_sum(h3, transmitters_index, num_segments=nx)
    cnt = jax.ops.segment_sum(jnp.ones((nn,), dtype=jnp.float32),
                              transmitters_index, num_segments=nx)
    return sums / jnp.maximum(cnt, 1.0)[:, None]
